# Initial kernel scaffold; baseline (speedup 1.0000x reference)
import functools

import jax
import jax.numpy as jnp
from jax import lax
from jax.experimental import pallas as pl
from jax.experimental.pallas import tpu as pltpu
from jax.experimental.pallas import tpu_sc as plsc

N_NODES = 10000
N_EDGES = 320000
D = 128
AD = 16
EPS = 1e-5

NC = 2
NS = 16
NW = NC * NS
CHUNK = 128
NCHUNKS = N_EDGES // CHUNK
ITERS_B = -(-NCHUNKS // NW)
CHUNKS_PER_SC = NCHUNKS // NC
ITERS_E = -(-CHUNKS_PER_SC // NS)

_mesh = plsc.VectorSubcoreMesh(core_axis_name="c", subcore_axis_name="s")


def _proj_body(x_ref, wfd_ref, wcd_ref, wfs_ref, wcs_ref,
               fd_ref, cd_ref, fs_ref, cs_ref):
    xv = x_ref[...]
    fd_ref[...] = jnp.dot(xv, wfd_ref[...], preferred_element_type=jnp.float32)
    cd_ref[...] = jnp.dot(xv, wcd_ref[...], preferred_element_type=jnp.float32)
    fs_ref[...] = jnp.dot(xv, wfs_ref[...], preferred_element_type=jnp.float32)
    cs_ref[...] = jnp.dot(xv, wcs_ref[...], preferred_element_type=jnp.float32)


def _project(x, wfd, wcd, wfs, wcs):
    blk = 1000
    w_spec = pl.BlockSpec((D, D), lambda i: (0, 0))
    o_spec = pl.BlockSpec((blk, D), lambda i: (i, 0))
    o_shape = jax.ShapeDtypeStruct((N_NODES, D), jnp.float32)
    return pl.pallas_call(
        _proj_body,
        grid=(N_NODES // blk,),
        in_specs=[pl.BlockSpec((blk, D), lambda i: (i, 0)),
                  w_spec, w_spec, w_spec, w_spec],
        out_specs=[o_spec, o_spec, o_spec, o_spec],
        out_shape=[o_shape, o_shape, o_shape, o_shape],
    )(x, wfd, wcd, wfs, wcs)


@functools.partial(
    pl.kernel,
    mesh=_mesh,
    out_type=[jax.ShapeDtypeStruct((N_EDGES, D), jnp.float32),
              jax.ShapeDtypeStruct((N_EDGES, D), jnp.float32)],
    scratch_types=[
        pltpu.VMEM((CHUNK,), jnp.int32),
        pltpu.VMEM((CHUNK,), jnp.int32),
        pltpu.VMEM((CHUNK,), jnp.int32),
        pltpu.VMEM((CHUNK, D), jnp.float32),
        pltpu.VMEM((CHUNK, D), jnp.float32),
        pltpu.VMEM((CHUNK, D), jnp.float32),
        pltpu.VMEM((CHUNK, D), jnp.float32),
        pltpu.VMEM_SHARED((NS * CHUNK, D), jnp.float32),
        pltpu.VMEM_SHARED((NS * CHUNK, D), jnp.float32),
        pltpu.SemaphoreType.DMA,
    ],
)
def _sc_gather_u(dst_h, src_h, fd_h, cd_h, fs_h, cs_h, rowid_h,
                 uf_h, uc_h,
                 idxd_v, idxs_v, myidx_v, bfd_v, bcd_v, bfs_v, bcs_v,
                 stf_s, stc_s, sem):
    cid = lax.axis_index("c")
    sid = lax.axis_index("s")
    w = sid * NC + cid
    base = sid * CHUNK
    pltpu.sync_copy(rowid_h.at[sid], myidx_v)

    def body(i, carry):
        ch = w + i * NW

        @pl.when(ch < NCHUNKS)
        def _():
            e0 = ch * CHUNK
            pltpu.sync_copy(dst_h.at[pl.ds(e0, CHUNK)], idxd_v)
            pltpu.sync_copy(src_h.at[pl.ds(e0, CHUNK)], idxs_v)
            cp1 = pltpu.async_copy(fd_h.at[idxd_v], bfd_v, sem)
            cp2 = pltpu.async_copy(cd_h.at[idxd_v], bcd_v, sem)
            cp3 = pltpu.async_copy(fs_h.at[idxs_v], bfs_v, sem)
            cp4 = pltpu.async_copy(cs_h.at[idxs_v], bcs_v, sem)
            cp1.wait()
            cp2.wait()
            cp3.wait()
            cp4.wait()
            pltpu.sync_copy(bfd_v, stf_s.at[pl.ds(base, CHUNK)])
            pltpu.sync_copy(bcd_v, stc_s.at[pl.ds(base, CHUNK)])
            pltpu.sync_copy(bfs_v, stf_s.at[myidx_v], add=True)
            pltpu.sync_copy(bcs_v, stc_s.at[myidx_v], add=True)
            pltpu.sync_copy(stf_s.at[pl.ds(base, CHUNK)],
                            uf_h.at[pl.ds(e0, CHUNK)])
            pltpu.sync_copy(stc_s.at[pl.ds(base, CHUNK)],
                            uc_h.at[pl.ds(e0, CHUNK)])
        return carry

    lax.fori_loop(0, ITERS_B, body, 0)


def _stats_body(uf_ref, uc_ref, a_ref, wef_ref, wec_ref,
                sf_ref, qf_ref, sc_ref, qc_ref):
    i = pl.program_id(0)

    @pl.when(i == 0)
    def _():
        sf_ref[...] = jnp.zeros_like(sf_ref)
        qf_ref[...] = jnp.zeros_like(qf_ref)
        sc_ref[...] = jnp.zeros_like(sc_ref)
        qc_ref[...] = jnp.zeros_like(qc_ref)

    av = a_ref[...]
    hf = uf_ref[...] + jnp.dot(av, wef_ref[...],
                               preferred_element_type=jnp.float32)
    hc = uc_ref[...] + jnp.dot(av, wec_ref[...],
                               preferred_element_type=jnp.float32)
    sf_ref[...] += jnp.sum(hf, axis=0, keepdims=True)
    qf_ref[...] += jnp.sum(hf * hf, axis=0, keepdims=True)
    sc_ref[...] += jnp.sum(hc, axis=0, keepdims=True)
    qc_ref[...] += jnp.sum(hc * hc, axis=0, keepdims=True)


def _stats(uf, uc, edge_attr, wef, wec):
    blk = 2000
    r_spec = pl.BlockSpec((1, D), lambda i: (0, 0))
    r_shape = jax.ShapeDtypeStruct((1, D), jnp.float32)
    return pl.pallas_call(
        _stats_body,
        grid=(N_EDGES // blk,),
        in_specs=[pl.BlockSpec((blk, D), lambda i: (i, 0)),
                  pl.BlockSpec((blk, D), lambda i: (i, 0)),
                  pl.BlockSpec((blk, AD), lambda i: (i, 0)),
                  pl.BlockSpec((AD, D), lambda i: (0, 0)),
                  pl.BlockSpec((AD, D), lambda i: (0, 0))],
        out_specs=[r_spec, r_spec, r_spec, r_spec],
        out_shape=[r_shape, r_shape, r_shape, r_shape],
    )(uf, uc, edge_attr, wef, wec)


def _msg_body(uf_ref, uc_ref, a_ref, wef_ref, wec_ref,
              scf_ref, shf_ref, scc_ref, shc_ref, m_ref):
    av = a_ref[...]
    hf = (uf_ref[...] + jnp.dot(av, wef_ref[...],
                                preferred_element_type=jnp.float32)
          ) * scf_ref[...] + shf_ref[...]
    hc = (uc_ref[...] + jnp.dot(av, wec_ref[...],
                                preferred_element_type=jnp.float32)
          ) * scc_ref[...] + shc_ref[...]
    m_ref[...] = jax.nn.sigmoid(hf) * jax.nn.softplus(hc)


def _msg(uf, uc, edge_attr, wef, wec, scf, shf, scc, shc):
    blk = 2000
    v_spec = pl.BlockSpec((1, D), lambda i: (0, 0))
    return pl.pallas_call(
        _msg_body,
        grid=(N_EDGES // blk,),
        in_specs=[pl.BlockSpec((blk, D), lambda i: (i, 0)),
                  pl.BlockSpec((blk, D), lambda i: (i, 0)),
                  pl.BlockSpec((blk, AD), lambda i: (i, 0)),
                  pl.BlockSpec((AD, D), lambda i: (0, 0)),
                  pl.BlockSpec((AD, D), lambda i: (0, 0)),
                  v_spec, v_spec, v_spec, v_spec],
        out_specs=pl.BlockSpec((blk, D), lambda i: (i, 0)),
        out_shape=jax.ShapeDtypeStruct((N_EDGES, D), jnp.float32),
    )(uf, uc, edge_attr, wef, wec, scf, shf, scc, shc)


_ROWBLK = 1000


@functools.partial(
    pl.kernel,
    mesh=_mesh,
    out_type=jax.ShapeDtypeStruct((NC * N_NODES, D), jnp.float32),
    scratch_types=[
        pltpu.VMEM((CHUNK,), jnp.int32),
        pltpu.VMEM((CHUNK, D), jnp.float32),
        pltpu.VMEM_SHARED((N_NODES, D), jnp.float32),
    ],
)
def _sc_scatter(dst_h, msg_h, zeros_h, part_h, idx_v, buf_v, acc_s):
    cid = lax.axis_index("c")
    sid = lax.axis_index("s")

    @pl.when(sid < N_NODES // _ROWBLK)
    def _():
        r0 = sid * _ROWBLK
        pltpu.sync_copy(zeros_h.at[pl.ds(r0, _ROWBLK)],
                        acc_s.at[pl.ds(r0, _ROWBLK)])

    plsc.subcore_barrier()
    lo = cid * CHUNKS_PER_SC

    def body(i, carry):
        ch = lo + sid + i * NS

        @pl.when(ch < lo + CHUNKS_PER_SC)
        def _():
            e0 = ch * CHUNK
            pltpu.sync_copy(dst_h.at[pl.ds(e0, CHUNK)], idx_v)
            pltpu.sync_copy(msg_h.at[pl.ds(e0, CHUNK)], buf_v)
            pltpu.sync_copy(buf_v, acc_s.at[idx_v], add=True)
        return carry

    lax.fori_loop(0, ITERS_E, body, 0)
    plsc.subcore_barrier()

    @pl.when(sid < N_NODES // _ROWBLK)
    def _():
        r0 = sid * _ROWBLK
        pltpu.sync_copy(acc_s.at[pl.ds(r0, _ROWBLK)],
                        part_h.at[pl.ds(cid * N_NODES + r0, _ROWBLK)])


def _final_body(p_ref, x_ref, g2_ref, b2_ref, o_ref):
    nf = p_ref[:N_NODES, :] + p_ref[N_NODES:, :]
    mu = jnp.mean(nf, axis=0, keepdims=True)
    cent = nf - mu
    var = jnp.mean(cent * cent, axis=0, keepdims=True)
    hn = cent * lax.rsqrt(var + EPS) * g2_ref[...] + b2_ref[...]
    o_ref[...] = jax.nn.softplus(hn + x_ref[...])


def _final(parts, x, g2, b2):
    return pl.pallas_call(
        _final_body,
        out_shape=jax.ShapeDtypeStruct((N_NODES, D), jnp.float32),
    )(parts, x, g2, b2)


def kernel(x, edge_index, edge_attr, W, b, gamma1, beta1, gamma2, beta2):
    src = edge_index[0]
    dst = edge_index[1]
    wfd = W[:D, :D].T
    wcd = W[D:, :D].T
    wfs = W[:D, D:2 * D].T
    wcs = W[D:, D:2 * D].T
    wef = W[:D, 2 * D:].T
    wec = W[D:, 2 * D:].T

    fd, cd, fs, cs = _project(x, wfd, wcd, wfs, wcs)

    rowid = jnp.arange(NS * CHUNK, dtype=jnp.int32).reshape(NS, CHUNK)
    uf, uc = _sc_gather_u(dst, src, fd, cd, fs, cs, rowid)

    sf, qf, sc_, qc = _stats(uf, uc, edge_attr, wef, wec)
    n = jnp.float32(N_EDGES)
    mf, mc = sf[0] / n, sc_[0] / n
    vf = qf[0] / n - mf * mf
    vc = qc[0] / n - mc * mc
    scf = gamma1[:D] * lax.rsqrt(vf + EPS)
    scc = gamma1[D:] * lax.rsqrt(vc + EPS)
    shf = beta1[:D] - mf * scf
    shc = beta1[D:] - mc * scc

    msg = _msg(uf, uc, edge_attr, wef, wec,
               scf[None, :], shf[None, :], scc[None, :], shc[None, :])

    zeros = jnp.zeros((N_NODES, D), jnp.float32)
    parts = _sc_scatter(dst, msg, zeros)

    return _final(parts, x, gamma2[None, :], beta2[None, :])


# speedup vs baseline: 2.3996x; 2.3996x over previous
import functools

import jax
import jax.numpy as jnp
from jax import lax
from jax.experimental import pallas as pl
from jax.experimental.pallas import tpu as pltpu
from jax.experimental.pallas import tpu_sc as plsc

N_NODES = 10000
N_EDGES = 320000
D = 128
AD = 16
EPS = 1e-5

NC = 2
NS = 16
NW = NC * NS
CHUNK = 128
NCHUNKS = N_EDGES // CHUNK
ITERS_B = -(-NCHUNKS // NW)
CHUNKS_PER_SC = NCHUNKS // NC
ITERS_E = -(-CHUNKS_PER_SC // NS)

def _proj_body(x_ref, wfd_ref, wcd_ref, wfs_ref, wcs_ref,
               fd_ref, cd_ref, fs_ref, cs_ref):
    xv = x_ref[...]
    fd_ref[...] = jnp.dot(xv, wfd_ref[...], preferred_element_type=jnp.float32)
    cd_ref[...] = jnp.dot(xv, wcd_ref[...], preferred_element_type=jnp.float32)
    fs_ref[...] = jnp.dot(xv, wfs_ref[...], preferred_element_type=jnp.float32)
    cs_ref[...] = jnp.dot(xv, wcs_ref[...], preferred_element_type=jnp.float32)


def _project(x, wfd, wcd, wfs, wcs):
    blk = 1000
    w_spec = pl.BlockSpec((D, D), lambda i: (0, 0))
    o_spec = pl.BlockSpec((blk, D), lambda i: (i, 0))
    o_shape = jax.ShapeDtypeStruct((N_NODES, D), jnp.float32)
    return pl.pallas_call(
        _proj_body,
        grid=(N_NODES // blk,),
        in_specs=[pl.BlockSpec((blk, D), lambda i: (i, 0)),
                  w_spec, w_spec, w_spec, w_spec],
        out_specs=[o_spec, o_spec, o_spec, o_spec],
        out_shape=[o_shape, o_shape, o_shape, o_shape],
    )(x, wfd, wcd, wfs, wcs)


@functools.cache
def _build_sc_gather_u():
  mesh = plsc.VectorSubcoreMesh(core_axis_name="c", subcore_axis_name="s")

  @functools.partial(
      pl.kernel,
      mesh=mesh,
      out_type=[jax.ShapeDtypeStruct((N_EDGES, D), jnp.float32),
                jax.ShapeDtypeStruct((N_EDGES, D), jnp.float32)],
      scratch_types=[
          pltpu.VMEM((CHUNK,), jnp.int32),
          pltpu.VMEM((CHUNK,), jnp.int32),
          pltpu.VMEM((CHUNK,), jnp.int32),
          pltpu.VMEM((CHUNK, D), jnp.float32),
          pltpu.VMEM((CHUNK, D), jnp.float32),
          pltpu.VMEM((CHUNK, D), jnp.float32),
          pltpu.VMEM((CHUNK, D), jnp.float32),
          pltpu.VMEM_SHARED((NS * CHUNK, D), jnp.float32),
          pltpu.VMEM_SHARED((NS * CHUNK, D), jnp.float32),
          pltpu.SemaphoreType.DMA,
      ],
  )
  def _sc_gather_u(dst_h, src_h, fd_h, cd_h, fs_h, cs_h, rowid_h,
                   uf_h, uc_h,
                   idxd_v, idxs_v, myidx_v, bfd_v, bcd_v, bfs_v, bcs_v,
                   stf_s, stc_s, sem):
    cid = lax.axis_index("c")
    sid = lax.axis_index("s")
    w = sid * NC + cid
    base = sid * CHUNK
    pltpu.sync_copy(rowid_h.at[sid], myidx_v)

    def body(i, carry):
        ch = w + i * NW

        @pl.when(ch < NCHUNKS)
        def _():
            e0 = ch * CHUNK
            pltpu.sync_copy(dst_h.at[pl.ds(e0, CHUNK)], idxd_v)
            pltpu.sync_copy(src_h.at[pl.ds(e0, CHUNK)], idxs_v)
            cp1 = pltpu.async_copy(fd_h.at[idxd_v], bfd_v, sem)
            cp2 = pltpu.async_copy(cd_h.at[idxd_v], bcd_v, sem)
            cp3 = pltpu.async_copy(fs_h.at[idxs_v], bfs_v, sem)
            cp4 = pltpu.async_copy(cs_h.at[idxs_v], bcs_v, sem)
            cp1.wait()
            cp2.wait()
            cp3.wait()
            cp4.wait()
            pltpu.sync_copy(bfd_v, stf_s.at[pl.ds(base, CHUNK)])
            pltpu.sync_copy(bcd_v, stc_s.at[pl.ds(base, CHUNK)])
            pltpu.sync_copy(bfs_v, stf_s.at[myidx_v], add=True)
            pltpu.sync_copy(bcs_v, stc_s.at[myidx_v], add=True)
            pltpu.sync_copy(stf_s.at[pl.ds(base, CHUNK)],
                            uf_h.at[pl.ds(e0, CHUNK)])
            pltpu.sync_copy(stc_s.at[pl.ds(base, CHUNK)],
                            uc_h.at[pl.ds(e0, CHUNK)])
        return carry

    lax.fori_loop(0, ITERS_B, body, 0)

  return _sc_gather_u


def _stats_body(uf_ref, uc_ref, a_ref, wef_ref, wec_ref,
                sf_ref, qf_ref, sc_ref, qc_ref):
    i = pl.program_id(0)

    @pl.when(i == 0)
    def _():
        sf_ref[...] = jnp.zeros_like(sf_ref)
        qf_ref[...] = jnp.zeros_like(qf_ref)
        sc_ref[...] = jnp.zeros_like(sc_ref)
        qc_ref[...] = jnp.zeros_like(qc_ref)

    av = a_ref[...]
    hf = uf_ref[...] + jnp.dot(av, wef_ref[...],
                               preferred_element_type=jnp.float32)
    hc = uc_ref[...] + jnp.dot(av, wec_ref[...],
                               preferred_element_type=jnp.float32)
    sf_ref[...] += jnp.sum(hf, axis=0, keepdims=True)
    qf_ref[...] += jnp.sum(hf * hf, axis=0, keepdims=True)
    sc_ref[...] += jnp.sum(hc, axis=0, keepdims=True)
    qc_ref[...] += jnp.sum(hc * hc, axis=0, keepdims=True)


def _stats(uf, uc, edge_attr, wef, wec):
    blk = 2000
    r_spec = pl.BlockSpec((1, D), lambda i: (0, 0))
    r_shape = jax.ShapeDtypeStruct((1, D), jnp.float32)
    return pl.pallas_call(
        _stats_body,
        grid=(N_EDGES // blk,),
        in_specs=[pl.BlockSpec((blk, D), lambda i: (i, 0)),
                  pl.BlockSpec((blk, D), lambda i: (i, 0)),
                  pl.BlockSpec((blk, AD), lambda i: (i, 0)),
                  pl.BlockSpec((AD, D), lambda i: (0, 0)),
                  pl.BlockSpec((AD, D), lambda i: (0, 0))],
        out_specs=[r_spec, r_spec, r_spec, r_spec],
        out_shape=[r_shape, r_shape, r_shape, r_shape],
    )(uf, uc, edge_attr, wef, wec)


def _msg_body(uf_ref, uc_ref, a_ref, wef_ref, wec_ref,
              scf_ref, shf_ref, scc_ref, shc_ref, m_ref):
    av = a_ref[...]
    hf = (uf_ref[...] + jnp.dot(av, wef_ref[...],
                                preferred_element_type=jnp.float32)
          ) * scf_ref[...] + shf_ref[...]
    hc = (uc_ref[...] + jnp.dot(av, wec_ref[...],
                                preferred_element_type=jnp.float32)
          ) * scc_ref[...] + shc_ref[...]
    m_ref[...] = jax.nn.sigmoid(hf) * jax.nn.softplus(hc)


def _msg(uf, uc, edge_attr, wef, wec, scf, shf, scc, shc):
    blk = 2000
    v_spec = pl.BlockSpec((1, D), lambda i: (0, 0))
    return pl.pallas_call(
        _msg_body,
        grid=(N_EDGES // blk,),
        in_specs=[pl.BlockSpec((blk, D), lambda i: (i, 0)),
                  pl.BlockSpec((blk, D), lambda i: (i, 0)),
                  pl.BlockSpec((blk, AD), lambda i: (i, 0)),
                  pl.BlockSpec((AD, D), lambda i: (0, 0)),
                  pl.BlockSpec((AD, D), lambda i: (0, 0)),
                  v_spec, v_spec, v_spec, v_spec],
        out_specs=pl.BlockSpec((blk, D), lambda i: (i, 0)),
        out_shape=jax.ShapeDtypeStruct((N_EDGES, D), jnp.float32),
    )(uf, uc, edge_attr, wef, wec, scf, shf, scc, shc)


_ROWBLK = 1000


@functools.cache
def _build_sc_scatter():
  mesh = plsc.VectorSubcoreMesh(core_axis_name="c", subcore_axis_name="s")

  @functools.partial(
      pl.kernel,
      mesh=mesh,
      out_type=jax.ShapeDtypeStruct((NC * N_NODES, D), jnp.float32),
      scratch_types=[
          pltpu.VMEM((CHUNK,), jnp.int32),
          pltpu.VMEM((CHUNK, D), jnp.float32),
          pltpu.VMEM_SHARED((N_NODES, D), jnp.float32),
      ],
  )
  def _sc_scatter(dst_h, msg_h, zeros_h, part_h, idx_v, buf_v, acc_s):
    cid = lax.axis_index("c")
    sid = lax.axis_index("s")

    @pl.when(sid < N_NODES // _ROWBLK)
    def _():
        r0 = sid * _ROWBLK
        pltpu.sync_copy(zeros_h.at[pl.ds(r0, _ROWBLK)],
                        acc_s.at[pl.ds(r0, _ROWBLK)])

    plsc.subcore_barrier()
    lo = cid * CHUNKS_PER_SC

    def body(i, carry):
        ch = lo + sid + i * NS

        @pl.when(ch < lo + CHUNKS_PER_SC)
        def _():
            e0 = ch * CHUNK
            pltpu.sync_copy(dst_h.at[pl.ds(e0, CHUNK)], idx_v)
            pltpu.sync_copy(msg_h.at[pl.ds(e0, CHUNK)], buf_v)
            pltpu.sync_copy(buf_v, acc_s.at[idx_v], add=True)
        return carry

    lax.fori_loop(0, ITERS_E, body, 0)
    plsc.subcore_barrier()

    @pl.when(sid < N_NODES // _ROWBLK)
    def _():
        r0 = sid * _ROWBLK
        pltpu.sync_copy(acc_s.at[pl.ds(r0, _ROWBLK)],
                        part_h.at[pl.ds(cid * N_NODES + r0, _ROWBLK)])

  return _sc_scatter


def _final_body(p_ref, x_ref, g2_ref, b2_ref, o_ref):
    nf = p_ref[:N_NODES, :] + p_ref[N_NODES:, :]
    mu = jnp.mean(nf, axis=0, keepdims=True)
    cent = nf - mu
    var = jnp.mean(cent * cent, axis=0, keepdims=True)
    hn = cent * lax.rsqrt(var + EPS) * g2_ref[...] + b2_ref[...]
    o_ref[...] = jax.nn.softplus(hn + x_ref[...])


def _final(parts, x, g2, b2):
    return pl.pallas_call(
        _final_body,
        out_shape=jax.ShapeDtypeStruct((N_NODES, D), jnp.float32),
    )(parts, x, g2, b2)


def kernel(x, edge_index, edge_attr, W, b, gamma1, beta1, gamma2, beta2):
    src = edge_index[0]
    dst = edge_index[1]
    wfd = W[:D, :D].T
    wcd = W[D:, :D].T
    wfs = W[:D, D:2 * D].T
    wcs = W[D:, D:2 * D].T
    wef = W[:D, 2 * D:].T
    wec = W[D:, 2 * D:].T

    fd, cd, fs, cs = _project(x, wfd, wcd, wfs, wcs)

    rowid = jnp.arange(NS * CHUNK, dtype=jnp.int32).reshape(NS, CHUNK)
    uf, uc = _build_sc_gather_u()(dst, src, fd, cd, fs, cs, rowid)

    sf, qf, sc_, qc = _stats(uf, uc, edge_attr, wef, wec)
    n = jnp.float32(N_EDGES)
    mf, mc = sf[0] / n, sc_[0] / n
    vf = qf[0] / n - mf * mf
    vc = qc[0] / n - mc * mc
    scf = gamma1[:D] * lax.rsqrt(vf + EPS)
    scc = gamma1[D:] * lax.rsqrt(vc + EPS)
    shf = beta1[:D] - mf * scf
    shc = beta1[D:] - mc * scc

    msg = _msg(uf, uc, edge_attr, wef, wec,
               scf[None, :], shf[None, :], scc[None, :], shc[None, :])

    zeros = jnp.zeros((N_NODES, D), jnp.float32)
    parts = _build_sc_scatter()(dst, msg, zeros)

    return _final(parts, x, gamma2[None, :], beta2[None, :])
